# baseline (speedup 1.0000x reference)
import functools

import jax
import jax.numpy as jnp
import numpy as np
from jax import lax
from jax.experimental import pallas as pl
from jax.experimental.pallas import tpu as pltpu
from jax.experimental.pallas import tpu_sc as plsc

_B, _C, _T = 4, 1024, 4096

_masked_channels = np.array([
    31, 35, 45, 85, 99, 112, 121, 130, 139, 144, 148, 152, 176, 179, 188,
    189, 197, 257, 263, 268, 304, 309, 312, 315, 318, 325, 356, 366, 398,
    409, 410, 429, 446, 448, 462, 480, 487, 493, 495, 499, 501, 507, 516,
    517, 518, 520, 532, 538, 541, 543, 544, 552, 557, 567, 569, 575, 577,
    582, 591, 602, 605, 617, 649, 659, 707, 709, 712, 739, 748, 750, 753,
    762, 768, 780, 787, 790, 793, 799, 842, 846, 848, 854, 857, 864, 879,
    883, 893, 895, 901, 914, 934, 942, 955, 957, 973, 976, 981, 984, 999,
    1001, 1005, 1016], dtype=np.int64)

_NW = 32
_ROWS_PER_W = (_B * _C) // _NW
_K_PAD = 24
_CHUNK = 8
_NCHUNKS = _ROWS_PER_W // _CHUNK

_idx_np = np.zeros((_NW, _K_PAD), dtype=np.int32)
for _w in range(_NW):
    _lo, _hi = _w * _ROWS_PER_W, (_w + 1) * _ROWS_PER_W
    _b = _lo // _C
    _rows = [_b * _C + int(c) for c in _masked_channels
             if _lo <= _b * _C + int(c) < _hi]
    assert 0 < len(_rows) <= _K_PAD
    _idx_np[_w] = (_rows + [_rows[0]] * _K_PAD)[:_K_PAD]

_mesh = plsc.VectorSubcoreMesh(core_axis_name="c", subcore_axis_name="s")


@functools.partial(
    pl.kernel,
    mesh=_mesh,
    out_type=jax.ShapeDtypeStruct((_B * _C, _T), jnp.float32),
    scratch_types=[
        pltpu.VMEM((_CHUNK, _T), jnp.float32),
        pltpu.VMEM((_CHUNK, _T), jnp.float32),
        pltpu.VMEM((_CHUNK, _T), jnp.float32),
        pltpu.VMEM((_K_PAD // 8, 8), jnp.int32),
        pltpu.SemaphoreType.DMA,
        pltpu.SemaphoreType.DMA,
        pltpu.SemaphoreType.DMA,
        pltpu.SemaphoreType.DMA,
    ],
)
def _sc_masked_copy(x_hbm, idx_hbm, zeros_hbm, out_hbm,
                    buf0, buf1, zsrc, idx_v,
                    sem_in0, sem_in1, sem_out0, sem_out1):
    wid = lax.axis_index("s") * 2 + lax.axis_index("c")
    base = wid * _ROWS_PER_W
    pltpu.sync_copy(idx_hbm.at[wid], idx_v)
    pltpu.sync_copy(zeros_hbm, zsrc)
    sems = (sem_in0, sem_in1, sem_out0, sem_out1)
    out_h = [None, None, None, None]
    for k in range(_NCHUNKS):
        s = k % 4
        if out_h[s] is not None:
            out_h[s].wait()
        rows = pl.ds(base + k * _CHUNK, _CHUNK)
        out_h[s] = pltpu.async_copy(x_hbm.at[rows], out_hbm.at[rows], sems[s])
    for h in out_h:
        h.wait()
    for j in range(_K_PAD // 8):
        pltpu.sync_copy(zsrc, out_hbm.at[idx_v.at[j]])


def kernel(x):
    B, C, T = x.shape
    x2 = x.reshape(B * C, T)
    idx = jnp.asarray(_idx_np.reshape(_NW, _K_PAD // 8, 8))
    zeros = jnp.zeros((_CHUNK, T), jnp.float32)
    out = _sc_masked_copy(x2, idx, zeros)
    return out.reshape(B, C, T)


# speedup vs baseline: 28.4962x; 28.4962x over previous
import functools

import jax
import jax.numpy as jnp
import numpy as np
from jax import lax
from jax.experimental import pallas as pl
from jax.experimental.pallas import tpu as pltpu
from jax.experimental.pallas import tpu_sc as plsc

_B, _C, _T = 4, 1024, 4096

_masked_channels = np.array([
    31, 35, 45, 85, 99, 112, 121, 130, 139, 144, 148, 152, 176, 179, 188,
    189, 197, 257, 263, 268, 304, 309, 312, 315, 318, 325, 356, 366, 398,
    409, 410, 429, 446, 448, 462, 480, 487, 493, 495, 499, 501, 507, 516,
    517, 518, 520, 532, 538, 541, 543, 544, 552, 557, 567, 569, 575, 577,
    582, 591, 602, 605, 617, 649, 659, 707, 709, 712, 739, 748, 750, 753,
    762, 768, 780, 787, 790, 793, 799, 842, 846, 848, 854, 857, 864, 879,
    883, 893, 895, 901, 914, 934, 942, 955, 957, 973, 976, 981, 984, 999,
    1001, 1005, 1016], dtype=np.int64)

_NW = 32
_ROWS_PER_W = (_B * _C) // _NW
_CHUNK = 8
_NCHUNKS = _ROWS_PER_W // _CHUNK
_ZWAVE = 4
_MAX_WAVES = 5

_zidx_np = np.zeros((_NW, _MAX_WAVES, _ZWAVE), dtype=np.int32)
_wave_counts = []
for _w in range(_NW):
    _lo = _w * _ROWS_PER_W
    _b = _lo // _C
    _rows = [_b * _C + int(c) for c in _masked_channels
             if _lo <= _b * _C + int(c) < _lo + _ROWS_PER_W]
    _n_waves = -(-len(_rows) // _ZWAVE)
    assert 0 < _n_waves <= _MAX_WAVES
    _wave_counts.append(_n_waves)
    _padded = (_rows + [_rows[0]] * (_MAX_WAVES * _ZWAVE))[:_MAX_WAVES * _ZWAVE]
    _zidx_np[_w] = np.asarray(_padded, np.int32).reshape(_MAX_WAVES, _ZWAVE)
assert _wave_counts[:8] == _wave_counts[8:16] == _wave_counts[16:24] == _wave_counts[24:]
_PACKED_WAVES = 0
for _o in range(8):
    assert _wave_counts[_o] < 8
    _PACKED_WAVES |= _wave_counts[_o] << (3 * _o)

_mesh = plsc.VectorSubcoreMesh(core_axis_name="c", subcore_axis_name="s")


@functools.partial(
    pl.kernel,
    mesh=_mesh,
    out_type=jax.ShapeDtypeStruct((_B * _C, _T), jnp.float32),
    scratch_types=[
        pltpu.VMEM((_CHUNK, _T), jnp.float32),
        pltpu.VMEM((_CHUNK, _T), jnp.float32),
        pltpu.VMEM((_ZWAVE, _T), jnp.float32),
        pltpu.VMEM((_MAX_WAVES, _ZWAVE), jnp.int32),
        pltpu.SemaphoreType.DMA,
        pltpu.SemaphoreType.DMA,
        pltpu.SemaphoreType.DMA,
        pltpu.SemaphoreType.DMA,
    ],
)
def _sc_masked_copy(x_hbm, zidx_hbm, zeros_hbm, out_hbm,
                    buf0, buf1, zsrc, zidx_v,
                    sem_in0, sem_in1, sem_out0, sem_out1):
    wid = lax.axis_index("s") * 2 + lax.axis_index("c")
    base = wid * _ROWS_PER_W
    n_waves = lax.shift_right_logical(
        jnp.int32(_PACKED_WAVES), 3 * lax.rem(wid, 8)) & 7
    pltpu.sync_copy(zidx_hbm.at[wid], zidx_v)
    pltpu.sync_copy(zeros_hbm, zsrc)
    bufs = (buf0, buf1)
    sem_in = (sem_in0, sem_in1)
    sem_out = (sem_out0, sem_out1)
    out_h = [None, None]
    for k in range(_NCHUNKS):
        s = k % 2
        if out_h[s] is not None:
            out_h[s].wait()
        rows = pl.ds(base + k * _CHUNK, _CHUNK)
        pltpu.async_copy(x_hbm.at[rows], bufs[s], sem_in[s]).wait()
        out_h[s] = pltpu.async_copy(bufs[s], out_hbm.at[rows], sem_out[s])
    out_h[0].wait()
    out_h[1].wait()
    for j in range(_MAX_WAVES):
        @pl.when(j < n_waves)
        def _():
            pltpu.sync_copy(zsrc, out_hbm.at[zidx_v.at[j]])


def kernel(x):
    B, C, T = x.shape
    x2 = x.reshape(B * C, T)
    zidx = jnp.asarray(_zidx_np)
    zeros = jnp.zeros((_ZWAVE, T), jnp.float32)
    out = _sc_masked_copy(x2, zidx, zeros)
    return out.reshape(B, C, T)
